# Initial kernel scaffold; baseline (speedup 1.0000x reference)
import jax
import jax.numpy as jnp
from jax.experimental import pallas as pl


def kernel(img):
    raise NotImplementedError("write your pallas kernel here")


# speedup vs baseline: 2.0058x; 2.0058x over previous
import jax
import jax.numpy as jnp
import numpy as np
from jax.experimental import pallas as pl

_OFFS = ((-1, 1), (0, 1), (1, 1), (1, 0), (1, -1), (0, -1), (-1, -1), (-1, 0))
_H = 512
_W = 512
_T = _H * _W


def _lbp_body(img_ref, hist_ref, out_ref):
    x = img_ref[0]
    gray = 0.299 * x[0] + 0.587 * x[1] + 0.114 * x[2]
    grayb = gray.astype(jnp.bfloat16).astype(jnp.float32)
    pg = jnp.pad(grayb, ((1, 1), (1, 1)))
    code = jnp.zeros((_H, _W), jnp.float32)
    for k, (dy, dx) in enumerate(_OFFS):
        nb = jax.lax.slice(pg, (1 + dy, 1 + dx), (1 + dy + _H, 1 + dx + _W))
        code = code + jnp.where(nb - grayb >= 0.0, np.float32(2 ** k),
                                np.float32(0.0))

    mean2 = jnp.sum(code) / np.float32(_T)
    var2 = jnp.sum((code - mean2) ** 2) / np.float32(_T - 1)
    out_ref[0, 0] = (code - mean2) * jax.lax.rsqrt(var2)

    bins = jax.lax.broadcasted_iota(jnp.int32, (1, 256), 1)

    def body(c, hist):
        cf = c.astype(jnp.float32)
        s = jnp.sum(jnp.where(code == cf, np.float32(1.0), np.float32(0.0)))
        return hist + jnp.where(bins == c, s, np.float32(0.0))

    hist = jax.lax.fori_loop(0, 256, body, jnp.zeros((1, 256), jnp.float32))
    hmean = jnp.sum(hist) / np.float32(256.0)
    hvar = jnp.sum((hist - hmean) ** 2) / np.float32(255.0)
    hist_ref[0] = (hist - hmean) * jax.lax.rsqrt(hvar)


def kernel(img):
    n = img.shape[0]
    hist, out = pl.pallas_call(
        _lbp_body,
        grid=(n,),
        in_specs=[pl.BlockSpec((1, 3, _H, _W), lambda i: (i, 0, 0, 0))],
        out_specs=[
            pl.BlockSpec((1, 1, 256), lambda i: (i, 0, 0)),
            pl.BlockSpec((1, 1, _H, _W), lambda i: (i, 0, 0, 0)),
        ],
        out_shape=[
            jax.ShapeDtypeStruct((n, 1, 256), jnp.float32),
            jax.ShapeDtypeStruct((n, 1, _H, _W), jnp.float32),
        ],
    )(img)
    return hist.reshape(n, 256), out


# speedup vs baseline: 18.0058x; 8.9771x over previous
import functools

import jax
import jax.numpy as jnp
import numpy as np
from jax import lax
from jax.experimental import pallas as pl
from jax.experimental.pallas import tpu as pltpu
from jax.experimental.pallas import tpu_sc as plsc

_OFFS = ((-1, 1), (0, 1), (1, 1), (1, 0), (1, -1), (0, -1), (-1, -1), (-1, 0))
_H = 512
_W = 512
_T = _H * _W
_N = 8

_NTEC = 32
_SLICES = _NTEC // _N
_SLICE = _T // _SLICES
_UNROLL = 8


def _lbp_body(img_ref, out_ref, code_ref):
    x = img_ref[0]
    gray = 0.299 * x[0] + 0.587 * x[1] + 0.114 * x[2]
    grayb = gray.astype(jnp.bfloat16).astype(jnp.float32)
    pg = jnp.pad(grayb, ((1, 1), (1, 1)))
    code = jnp.zeros((_H, _W), jnp.float32)
    for k, (dy, dx) in enumerate(_OFFS):
        nb = jax.lax.slice(pg, (1 + dy, 1 + dx), (1 + dy + _H, 1 + dx + _W))
        code = code + jnp.where(nb - grayb >= 0.0, np.float32(2 ** k),
                                np.float32(0.0))

    code_ref[0] = code.astype(jnp.int32)

    mean2 = jnp.sum(code) / np.float32(_T)
    var2 = jnp.sum((code - mean2) ** 2) / np.float32(_T - 1)
    out_ref[0, 0] = (code - mean2) * jax.lax.rsqrt(var2)


def _sc_hist_body(codes_hbm, out_hbm, buf, hist, acc):
    c = lax.axis_index("c")
    s = lax.axis_index("s")
    wid = s * 2 + c
    n = wid % _N
    sl = wid // _N
    base = n * _T + sl * _SLICE

    pltpu.sync_copy(codes_hbm.at[pl.ds(base, _SLICE)], buf)

    zeros = jnp.zeros((16,), jnp.float32)
    for j in range(256):
        hist[pl.ds(j * 16, 16)] = zeros

    lanes = lax.iota(jnp.int32, 16) * 256
    ones = jnp.ones((16,), jnp.float32)

    def inner(i, carry):
        for u in range(_UNROLL):
            idx = buf[pl.ds((i * _UNROLL + u) * 16, 16)] + lanes
            plsc.addupdate_scatter(hist, [idx], ones)
        return carry

    lax.fori_loop(0, _SLICE // (16 * _UNROLL), inner, 0)

    for j in range(16):
        a = hist[pl.ds(j * 16, 16)]
        for r in range(1, 16):
            a = a + hist[pl.ds(r * 256 + j * 16, 16)]
        acc[pl.ds(j * 16, 16)] = a

    pltpu.sync_copy(acc, out_hbm.at[wid])


def _sc_hist(codes_flat):
    return pl.kernel(
        _sc_hist_body,
        out_type=jax.ShapeDtypeStruct((_NTEC, 256), jnp.float32),
        mesh=plsc.VectorSubcoreMesh(core_axis_name="c", subcore_axis_name="s",
                                    num_cores=2, num_subcores=16),
        scratch_types=[
            pltpu.VMEM((_SLICE,), jnp.int32),
            pltpu.VMEM((16 * 256,), jnp.float32),
            pltpu.VMEM((256,), jnp.float32),
        ],
        compiler_params=pltpu.CompilerParams(needs_layout_passes=False),
    )(codes_flat)


def _hist_norm_body(part_ref, hist_ref):
    h = part_ref[...].reshape(_SLICES, _N, 256).sum(axis=0)
    hmean = jnp.sum(h, axis=1, keepdims=True) / np.float32(256.0)
    hvar = jnp.sum((h - hmean) ** 2, axis=1, keepdims=True) / np.float32(255.0)
    hist_ref[...] = (h - hmean) * jax.lax.rsqrt(hvar)


def kernel(img):
    n = img.shape[0]
    out, codes = pl.pallas_call(
        _lbp_body,
        grid=(n,),
        in_specs=[pl.BlockSpec((1, 3, _H, _W), lambda i: (i, 0, 0, 0))],
        out_specs=[
            pl.BlockSpec((1, 1, _H, _W), lambda i: (i, 0, 0, 0)),
            pl.BlockSpec((1, _H, _W), lambda i: (i, 0, 0)),
        ],
        out_shape=[
            jax.ShapeDtypeStruct((n, 1, _H, _W), jnp.float32),
            jax.ShapeDtypeStruct((n, _H, _W), jnp.int32),
        ],
    )(img)

    parts = _sc_hist(codes.reshape(n * _T))

    hist = pl.pallas_call(
        _hist_norm_body,
        in_specs=[pl.BlockSpec((_NTEC, 256), lambda: (0, 0))],
        out_specs=pl.BlockSpec((_N, 256), lambda: (0, 0)),
        out_shape=jax.ShapeDtypeStruct((_N, 256), jnp.float32),
    )(parts)

    return hist, out


# speedup vs baseline: 20.6567x; 1.1472x over previous
import functools

import jax
import jax.numpy as jnp
import numpy as np
from jax import lax
from jax.experimental import pallas as pl
from jax.experimental.pallas import tpu as pltpu
from jax.experimental.pallas import tpu_sc as plsc

_OFFS = ((-1, 1), (0, 1), (1, 1), (1, 0), (1, -1), (0, -1), (-1, -1), (-1, 0))
_H = 512
_W = 512
_T = _H * _W
_N = 8

_NTEC = 32
_SLICES = _NTEC // _N
_SLICE = _T // _SLICES
_UNROLL = 16


def _lbp_body(img_ref, out_ref, code_ref):
    x = img_ref[0]
    gray = 0.299 * x[0] + 0.587 * x[1] + 0.114 * x[2]
    grayb = gray.astype(jnp.bfloat16).astype(jnp.float32)
    pg = jnp.pad(grayb, ((1, 1), (1, 1)))
    code = jnp.zeros((_H, _W), jnp.float32)
    for k, (dy, dx) in enumerate(_OFFS):
        nb = jax.lax.slice(pg, (1 + dy, 1 + dx), (1 + dy + _H, 1 + dx + _W))
        code = code + jnp.where(nb - grayb >= 0.0, np.float32(2 ** k),
                                np.float32(0.0))

    code_ref[0] = code.astype(jnp.int32).reshape(_T // 128, 128)

    mean2 = jnp.sum(code) / np.float32(_T)
    var2 = jnp.sum((code - mean2) ** 2) / np.float32(_T - 1)
    out_ref[0, 0] = (code - mean2) * jax.lax.rsqrt(var2)


def _sc_hist_body(codes_hbm, out_hbm, buf, hist, acc):
    c = lax.axis_index("c")
    s = lax.axis_index("s")
    wid = s * 2 + c
    n = wid % _N
    sl = wid // _N
    base = n * _T + sl * _SLICE

    pltpu.sync_copy(codes_hbm.at[pl.ds(base, _SLICE)], buf)

    zeros = jnp.zeros((16,), jnp.float32)
    for j in range(256):
        hist[pl.ds(j * 16, 16)] = zeros

    lanes = lax.iota(jnp.int32, 16)
    ones = jnp.ones((16,), jnp.float32)

    def inner(i, carry):
        for u in range(_UNROLL):
            idx = buf[pl.ds((i * _UNROLL + u) * 16, 16)] * 16 + lanes
            plsc.addupdate_scatter(hist, [idx], ones)
        return carry

    lax.fori_loop(0, _SLICE // (16 * _UNROLL), inner, 0)

    for j in range(16):
        a = jnp.zeros((16,), jnp.float32)
        for b in range(16):
            s = jnp.sum(hist[pl.ds((j * 16 + b) * 16, 16)])
            a = jnp.where(lanes == b, s, a)
        acc[pl.ds(j * 16, 16)] = a

    pltpu.sync_copy(acc, out_hbm.at[wid])


def _sc_hist(codes_flat):
    return pl.kernel(
        _sc_hist_body,
        out_type=jax.ShapeDtypeStruct((_NTEC, 256), jnp.float32),
        mesh=plsc.VectorSubcoreMesh(core_axis_name="c", subcore_axis_name="s",
                                    num_cores=2, num_subcores=16),
        scratch_types=[
            pltpu.VMEM((_SLICE,), jnp.int32),
            pltpu.VMEM((16 * 256,), jnp.float32),
            pltpu.VMEM((256,), jnp.float32),
        ],
        compiler_params=pltpu.CompilerParams(needs_layout_passes=False),
    )(codes_flat)


def _hist_norm_body(part_ref, hist_ref):
    h = part_ref[...].reshape(_SLICES, _N, 256).sum(axis=0)
    hmean = jnp.sum(h, axis=1, keepdims=True) / np.float32(256.0)
    hvar = jnp.sum((h - hmean) ** 2, axis=1, keepdims=True) / np.float32(255.0)
    hist_ref[...] = (h - hmean) * jax.lax.rsqrt(hvar)


def kernel(img):
    n = img.shape[0]
    out, codes = pl.pallas_call(
        _lbp_body,
        grid=(n,),
        in_specs=[pl.BlockSpec((1, 3, _H, _W), lambda i: (i, 0, 0, 0))],
        out_specs=[
            pl.BlockSpec((1, 1, _H, _W), lambda i: (i, 0, 0, 0)),
            pl.BlockSpec((1, _T // 128, 128), lambda i: (i, 0, 0)),
        ],
        out_shape=[
            jax.ShapeDtypeStruct((n, 1, _H, _W), jnp.float32),
            jax.ShapeDtypeStruct((n, _T // 128, 128), jnp.int32),
        ],
    )(img)

    parts = _sc_hist(codes.reshape(n * _T))

    hist = pl.pallas_call(
        _hist_norm_body,
        in_specs=[pl.BlockSpec((_NTEC, 256), lambda: (0, 0))],
        out_specs=pl.BlockSpec((_N, 256), lambda: (0, 0)),
        out_shape=jax.ShapeDtypeStruct((_N, 256), jnp.float32),
    )(parts)

    return hist, out
